# baseline (speedup 1.0000x reference)
import jax
import jax.numpy as jnp
from jax import lax
from jax.experimental import pallas as pl
from jax.experimental.pallas import tpu as pltpu
from jax.experimental.pallas import tpu_sc as plsc

RAW = 512
REG = 32
GATHER = 16
L = 16

CH = 64
NBUF = 4

_NC = 2
_NS = 16
_NW = _NC * _NS


def _sc_body(data_hbm, proto_hbm, out_hbm, proto_v, feat_v, z24_v, bufs,
             gsems, ssems, rsem):
    B = data_hbm.shape[0]
    bpw = B // _NW
    wid = lax.axis_index("s") * _NC + lax.axis_index("c")
    b0 = wid * bpw

    C = data_hbm.shape[2]
    nchunk = C // L
    cpb = RAW // CH
    steps = bpw * cpb

    def src_at(t):
        b = b0 + t // cpb
        r = (t % cpb) * CH
        return data_hbm.at[pl.ds(b, 1), pl.ds(r, CH)]

    def dst_at(t):
        b = b0 + t // cpb
        r = (t % cpb) * CH
        return out_hbm.at[pl.ds(b, 1), pl.ds(r, CH)]

    for s in range(NBUF):
        pltpu.async_copy(src_at(s), bufs.at[s], gsems.at[s])

    pltpu.sync_copy(proto_hbm, proto_v)

    zero16 = jnp.zeros((L,), jnp.float32)

    def zrow24(r, carry):
        for k in range(nchunk):
            z24_v[0, r, pl.ds(k * L, L)] = zero16
        return carry

    lax.fori_loop(0, REG - 8, zrow24, 0)

    def zrowf(i, carry):
        r = i // 7
        j = 1 + i % 7
        for k in range(nchunk):
            feat_v[r, 0, j, pl.ds(k * L, L)] = zero16
        return carry

    lax.fori_loop(0, bpw * 7, zrowf, 0)

    iota = lax.iota(jnp.int32, L)

    def attention(s, t):
        i = t // cpb
        b = b0 + i
        sims = zero16
        for j in range(GATHER):
            acc = zero16
            for k in range(nchunk):
                acc = acc + bufs[s, 0, j, pl.ds(k * L, L)] * proto_v[0, pl.ds(k * L, L)]
            sj = jnp.sum(acc) * (1.0 / 16.0)
            sims = jnp.where(iota == j, sj, sims)
        m = jnp.max(sims)
        e = jnp.exp(sims - m)
        attn = e / jnp.sum(e)
        for k in range(nchunk):
            acc = zero16
            for j in range(GATHER):
                acc = acc + attn[j] * bufs[s, 0, j, pl.ds(k * L, L)]
            feat_v[i, 0, 0, pl.ds(k * L, L)] = acc
        pltpu.async_copy(feat_v.at[i], out_hbm.at[pl.ds(b, 1), pl.ds(RAW, 8)], rsem)
        pltpu.async_copy(z24_v, out_hbm.at[pl.ds(b, 1), pl.ds(RAW + 8, REG - 8)], rsem)

    def ring_phase_a(g):
        for s in range(NBUF):
            t = g * NBUF + s
            pltpu.make_async_copy(src_at(t), bufs.at[s], gsems.at[s]).wait()
            pltpu.async_copy(bufs.at[s], dst_at(t), ssems.at[s])

            @pl.when(t % cpb == 0)
            def _():
                attention(s, t)

    def ring_round(g, carry):
        ring_phase_a(g)
        for s in range(NBUF):
            t = g * NBUF + s
            pltpu.make_async_copy(bufs.at[s], dst_at(t), ssems.at[s]).wait()
            pltpu.async_copy(src_at(t + NBUF), bufs.at[s], gsems.at[s])
        return carry

    lax.fori_loop(0, steps // NBUF - 1, ring_round, 0)

    ring_phase_a(steps // NBUF - 1)
    for s in range(NBUF):
        t = steps - NBUF + s
        pltpu.make_async_copy(bufs.at[s], dst_at(t), ssems.at[s]).wait()

    def rdrain(i, carry):
        b = b0 + i
        pltpu.make_async_copy(
            feat_v.at[0], out_hbm.at[pl.ds(b, 1), pl.ds(RAW, 8)], rsem
        ).wait()
        pltpu.make_async_copy(
            z24_v, out_hbm.at[pl.ds(b, 1), pl.ds(RAW + 8, REG - 8)], rsem
        ).wait()
        return carry

    lax.fori_loop(0, bpw, rdrain, 0)


@jax.jit
def kernel(data, region_prototypes):
    B, T, C = data.shape
    mesh = plsc.VectorSubcoreMesh(core_axis_name="c", subcore_axis_name="s")
    bpw = B // _NW
    run = pl.kernel(
        _sc_body,
        out_type=jax.ShapeDtypeStruct((B, T, C), data.dtype),
        mesh=mesh,
        compiler_params=pltpu.CompilerParams(needs_layout_passes=False),
        scratch_types=[
            pltpu.VMEM((REG, C), jnp.float32),
            pltpu.VMEM((bpw, 1, 8, C), jnp.float32),
            pltpu.VMEM((1, REG - 8, C), jnp.float32),
            pltpu.VMEM((NBUF, 1, CH, C), jnp.float32),
            pltpu.SemaphoreType.DMA((NBUF,)),
            pltpu.SemaphoreType.DMA((NBUF,)),
            pltpu.SemaphoreType.DMA,
        ],
    )
    return run(data, region_prototypes)


# speedup vs baseline: 1.0541x; 1.0541x over previous
import jax
import jax.numpy as jnp
from jax import lax
from jax.experimental import pallas as pl
from jax.experimental.pallas import tpu as pltpu
from jax.experimental.pallas import tpu_sc as plsc

RAW = 512
REG = 32
GATHER = 16
L = 16

CH = 128
NBUF = 2

_NC = 2
_NS = 16
_NW = _NC * _NS


def _sc_body(data_hbm, proto_hbm, out_hbm, x_v, proto_v, feat_v, z24_v, bufs,
             gsems, ssems, xsem, rsem):
    B = data_hbm.shape[0]
    bpw = B // _NW
    sid = lax.axis_index("s")
    wid = sid * _NC + lax.axis_index("c")
    b0 = wid * bpw

    C = data_hbm.shape[2]
    nchunk = C // L
    cpb = RAW // CH
    steps = bpw * cpb

    def src_at(t):
        b = b0 + t // cpb
        r = (t % cpb) * CH
        return data_hbm.at[pl.ds(b, 1), pl.ds(r, CH)]

    def dst_at(t):
        b = b0 + t // cpb
        r = (t % cpb) * CH
        return out_hbm.at[pl.ds(b, 1), pl.ds(r, CH)]

    def buf_at(s):
        return bufs.at[sid, s]

    for s in range(NBUF):
        pltpu.async_copy(src_at(s), buf_at(s), gsems.at[s])

    xcopy = pltpu.async_copy(
        data_hbm.at[pl.ds(b0, bpw), pl.ds(0, GATHER)], x_v, xsem
    )

    pltpu.sync_copy(proto_hbm, proto_v)

    zero16 = jnp.zeros((L,), jnp.float32)

    def zrow24(r, carry):
        for k in range(nchunk):
            z24_v[0, r, pl.ds(k * L, L)] = zero16
        return carry

    lax.fori_loop(0, REG - 8, zrow24, 0)

    def zrowf(i, carry):
        r = i // 7
        j = 1 + i % 7
        for k in range(nchunk):
            feat_v[r, 0, j, pl.ds(k * L, L)] = zero16
        return carry

    lax.fori_loop(0, bpw * 7, zrowf, 0)

    iota = lax.iota(jnp.int32, L)

    xcopy.wait()

    def batch_body(i, carry):
        b = b0 + i
        sims = zero16
        for j in range(GATHER):
            acc = zero16
            for k in range(nchunk):
                acc = acc + x_v[i, j, pl.ds(k * L, L)] * proto_v[0, pl.ds(k * L, L)]
            sj = jnp.sum(acc) * (1.0 / 16.0)
            sims = jnp.where(iota == j, sj, sims)
        m = jnp.max(sims)
        e = jnp.exp(sims - m)
        attn = e / jnp.sum(e)
        for k in range(nchunk):
            acc = zero16
            for j in range(GATHER):
                acc = acc + attn[j] * x_v[i, j, pl.ds(k * L, L)]
            feat_v[i, 0, 0, pl.ds(k * L, L)] = acc
        pltpu.async_copy(feat_v.at[i], out_hbm.at[pl.ds(b, 1), pl.ds(RAW, 8)], rsem)
        pltpu.async_copy(z24_v, out_hbm.at[pl.ds(b, 1), pl.ds(RAW + 8, REG - 8)], rsem)
        return carry

    lax.fori_loop(0, bpw, batch_body, 0)

    def slot_step(s, t, issue_next):
        pltpu.make_async_copy(src_at(t), buf_at(s), gsems.at[s]).wait()
        pltpu.async_copy(buf_at(s), dst_at(t), ssems.at[s])
        pltpu.make_async_copy(buf_at(s), dst_at(t), ssems.at[s]).wait()
        if issue_next:
            pltpu.async_copy(src_at(t + NBUF), buf_at(s), gsems.at[s])

    def ring_round(g, carry):
        for s in range(NBUF):
            slot_step(s, g * NBUF + s, True)
        return carry

    lax.fori_loop(0, steps // NBUF - 1, ring_round, 0)

    for s in range(NBUF):
        slot_step(s, steps - NBUF + s, False)

    def rdrain(i, carry):
        b = b0 + i
        pltpu.make_async_copy(
            feat_v.at[0], out_hbm.at[pl.ds(b, 1), pl.ds(RAW, 8)], rsem
        ).wait()
        pltpu.make_async_copy(
            z24_v, out_hbm.at[pl.ds(b, 1), pl.ds(RAW + 8, REG - 8)], rsem
        ).wait()
        return carry

    lax.fori_loop(0, bpw, rdrain, 0)


@jax.jit
def kernel(data, region_prototypes):
    B, T, C = data.shape
    mesh = plsc.VectorSubcoreMesh(core_axis_name="c", subcore_axis_name="s")
    bpw = B // _NW
    run = pl.kernel(
        _sc_body,
        out_type=jax.ShapeDtypeStruct((B, T, C), data.dtype),
        mesh=mesh,
        compiler_params=pltpu.CompilerParams(needs_layout_passes=False),
        scratch_types=[
            pltpu.VMEM((bpw, GATHER, C), jnp.float32),
            pltpu.VMEM((REG, C), jnp.float32),
            pltpu.VMEM((bpw, 1, 8, C), jnp.float32),
            pltpu.VMEM((1, REG - 8, C), jnp.float32),
            pltpu.VMEM_SHARED((_NS, NBUF, 1, CH, C), jnp.float32),
            pltpu.SemaphoreType.DMA((NBUF,)),
            pltpu.SemaphoreType.DMA((NBUF,)),
            pltpu.SemaphoreType.DMA,
            pltpu.SemaphoreType.DMA,
        ],
    )
    return run(data, region_prototypes)
